# baseline (speedup 1.0000x reference)
import dataclasses
import functools

import jax
import jax.numpy as jnp
from jax import lax
from jax.experimental import pallas as pl
from jax.experimental.pallas import tpu as pltpu
from jax.experimental.pallas import tpu_sc as plsc

T = 2048
C = 2048
DFF = 8192
E = 8
TM = 512
TF = 1024
GRID_F = DFF // TF
MAX_TILES = 16
ROWS = MAX_TILES * TM
NW = 16
PAIRS = 2 * T
PPW = PAIRS // NW
VPW = PPW // 16


def _router_body(x_ref, wg_ref, bg_ref, eids_ref, gates_ref):
    s = jnp.dot(x_ref[...], wg_ref[...], preferred_element_type=jnp.float32)
    s = s + bg_ref[...]
    lane = lax.broadcasted_iota(jnp.int32, s.shape, 1)
    m1 = jnp.max(s, axis=1, keepdims=True)
    e1 = jnp.min(jnp.where(s == m1, lane, E), axis=1, keepdims=True)
    s2 = jnp.where(lane == e1, -jnp.inf, s)
    m2 = jnp.max(s2, axis=1, keepdims=True)
    e2 = jnp.min(jnp.where(s2 == m2, lane, E), axis=1, keepdims=True)
    g2 = 1.0 / (1.0 + jnp.exp(m1 - m2))
    g1 = 1.0 - g2
    eids_ref[...] = jnp.concatenate([e1, e2], axis=1).astype(jnp.int32)
    gates_ref[...] = jnp.concatenate([g1, g2], axis=1)


def _router(x_flat, Wg, bg):
    return pl.pallas_call(
        _router_body,
        out_shape=(
            jax.ShapeDtypeStruct((T, 2), jnp.int32),
            jax.ShapeDtypeStruct((T, 2), jnp.float32),
        ),
        in_specs=[
            pl.BlockSpec((T, C), lambda: (0, 0)),
            pl.BlockSpec((C, E), lambda: (0, 0)),
            pl.BlockSpec((1, E), lambda: (0, 0)),
        ],
        out_specs=(
            pl.BlockSpec((T, 2), lambda: (0, 0)),
            pl.BlockSpec((T, 2), lambda: (0, 0)),
        ),
    )(x_flat, Wg, bg.reshape(1, E))


def _sortmeta_body(eids_hbm, pos0_hbm, pos1_hbm, meta_hbm,
                   eidv, rankv_m, histv, allh, shh, startv, outv):
    cid = lax.axis_index("c")
    wid = lax.axis_index("s")
    iota = lax.iota(jnp.int32, 16)

    @pl.when(cid == 0)
    def _work():
        pltpu.sync_copy(eids_hbm.at[pl.ds(wid * PPW, PPW)], eidv)

        def vbody(v, cnts):
            ev = eidv[pl.ds(v * 16, 16)]
            rk = jnp.zeros((16,), jnp.int32)
            new = []
            for e in range(E):
                m = ev == e
                mi = m.astype(jnp.int32)
                pref = plsc.cumsum(mi)
                tot = jnp.sum(mi)
                rk = jnp.where(m, cnts[e] + pref - 1, rk)
                new.append(cnts[e] + tot)
            rankv_m[pl.ds(v * 16, 16)] = rk
            return tuple(new)

        cnts = lax.fori_loop(0, VPW, vbody,
                             tuple(jnp.int32(0) for _ in range(E)))

        hv = jnp.zeros((16,), jnp.int32)
        for e in range(E):
            hv = jnp.where(iota == e, cnts[e], hv)
        histv[...] = hv
        pltpu.sync_copy(histv, shh.at[wid])
        plsc.subcore_barrier()

        pltpu.sync_copy(shh, allh)
        base = jnp.zeros((16,), jnp.int32)
        tot = jnp.zeros((16,), jnp.int32)
        for j in range(NW):
            hj = allh[j, :]
            base = base + jnp.where(j < wid, hj, 0)
            tot = tot + hj
        padded = ((tot + (TM - 1)) >> 9) << 9
        o_incl = plsc.cumsum(padded)
        o_excl = o_incl - padded
        startv[...] = o_excl + base

        def pbody(v, _):
            ev = eidv[pl.ds(v * 16, 16)]
            st = plsc.load_gather(startv, [ev])
            rankv_m[pl.ds(v * 16, 16)] = st + rankv_m[pl.ds(v * 16, 16)]
            return 0

        lax.fori_loop(0, VPW, pbody, 0)

        half = PPW // 2
        for u in range(VPW // 2):
            idx0 = u * 32 + 2 * iota
            outv[pl.ds(u * 16, 16)] = plsc.load_gather(rankv_m, [idx0])
            outv[pl.ds(half + u * 16, 16)] = plsc.load_gather(rankv_m, [idx0 + 1])
        pltpu.sync_copy(outv.at[pl.ds(0, half)],
                        pos0_hbm.at[pl.ds(wid * half, half)])
        pltpu.sync_copy(outv.at[pl.ds(half, half)],
                        pos1_hbm.at[pl.ds(wid * half, half)])

        @pl.when(wid == 0)
        def _meta():
            tev = jnp.zeros((16,), jnp.int32)
            for e in range(E):
                oe = jnp.sum(jnp.where(iota == e, o_incl, 0))
                tev = tev + (oe <= iota * TM).astype(jnp.int32)
            last_e = jnp.max(jnp.where(tot > 0, iota, 0))
            tev = jnp.minimum(tev, last_e)
            nt = jnp.sum(padded) >> 9
            histv[...] = tev
            pltpu.sync_copy(histv, meta_hbm.at[pl.ds(0, 16)])
            histv[...] = jnp.zeros((16,), jnp.int32) + nt
            pltpu.sync_copy(histv, meta_hbm.at[pl.ds(16, 16)])


def _sc_compiler_params():
    cp = pltpu.CompilerParams()
    if "needs_layout_passes" in pltpu.CompilerParams.__dataclass_fields__:
        cp = dataclasses.replace(cp, needs_layout_passes=False)
    return cp


def _sortmeta(eids_flat):
    mesh = plsc.VectorSubcoreMesh(core_axis_name="c", subcore_axis_name="s")
    kern = functools.partial(
        pl.kernel,
        compiler_params=_sc_compiler_params(),
        out_type=(
            jax.ShapeDtypeStruct((T,), jnp.int32),
            jax.ShapeDtypeStruct((T,), jnp.int32),
            jax.ShapeDtypeStruct((32,), jnp.int32),
        ),
        mesh=mesh,
        scratch_types=[
            pltpu.VMEM((PPW,), jnp.int32),
            pltpu.VMEM((PPW,), jnp.int32),
            pltpu.VMEM((16,), jnp.int32),
            pltpu.VMEM((NW, 16), jnp.int32),
            pltpu.VMEM_SHARED((NW, 16), jnp.int32),
            pltpu.VMEM((16,), jnp.int32),
            pltpu.VMEM((PPW,), jnp.int32),
        ],
    )(_sortmeta_body)
    return kern(eids_flat)


def _dispatch_body(x_hbm, pos0_hbm, pos1_hbm, xs_hbm, rows, i0, i1, sem):
    cid = lax.axis_index("c")
    sid = lax.axis_index("s")
    w = sid * 2 + cid
    tpw = T // 32
    for chunk in range(2):
        base = w * tpw + chunk * 32
        pltpu.sync_copy(x_hbm.at[pl.ds(base, 32)], rows)
        pltpu.sync_copy(pos0_hbm.at[pl.ds(base, 32)], i0)
        pltpu.sync_copy(pos1_hbm.at[pl.ds(base, 32)], i1)
        pltpu.async_copy(rows, xs_hbm.at[i0], sem).wait()
        pltpu.async_copy(rows, xs_hbm.at[i1], sem).wait()


def _dispatch(x_flat, pos0, pos1):
    mesh = plsc.VectorSubcoreMesh(core_axis_name="c", subcore_axis_name="s")
    kern = functools.partial(
        pl.kernel,
        out_type=jax.ShapeDtypeStruct((ROWS, C), jnp.float32),
        mesh=mesh,
        scratch_types=[
            pltpu.VMEM((32, C), jnp.float32),
            pltpu.VMEM((32,), jnp.int32),
            pltpu.VMEM((32,), jnp.int32),
            pltpu.SemaphoreType.DMA,
        ],
    )(_dispatch_body)
    return kern(x_flat, pos0, pos1)


KA = 8
KC = C // KA
KB = 16
TFB = DFF // KB
NS = KA + KB


def _gelu(h):
    return 0.5 * h * (1.0 + jnp.tanh(0.7978845608028654
                                     * (h + 0.044715 * h * h * h)))


def _ffn_body(meta_ref, xs_ref, wfc_ref, bfc_ref, wproj_ref, bproj_ref,
              o_ref, h_ref):
    j = pl.program_id(0)
    s = pl.program_id(1)
    nt = meta_ref[16]

    @pl.when(jnp.logical_and(j < nt, s < KA))
    def _phase_a():
        for q in range(4):
            lo, hi = q * (DFF // 4), (q + 1) * (DFF // 4)
            part = jnp.dot(xs_ref[...], wfc_ref[0, :, lo:hi],
                           preferred_element_type=jnp.float32)
            prev = jnp.where(s == 0, bfc_ref[0, :, lo:hi], h_ref[:, lo:hi])
            h_ref[:, lo:hi] = part + prev

    @pl.when(jnp.logical_and(j < nt, s >= KA))
    def _phase_b():
        for n in range(KB):
            @pl.when(s == KA + n)
            def _(n=n):
                hseg = _gelu(h_ref[:, n * TFB:(n + 1) * TFB])
                yb = jnp.dot(hseg, wproj_ref[0],
                             preferred_element_type=jnp.float32)
                if n == 0:
                    o_ref[...] = yb + bproj_ref[0]
                else:
                    o_ref[...] = o_ref[...] + yb

    @pl.when(jnp.logical_and(j >= nt, s == 0))
    def _inactive():
        o_ref[...] = jnp.zeros_like(o_ref)


def _ffn(meta, xs, Wfc, bfc, Wproj, bproj):
    def jpin(j, s, m):
        return jnp.minimum(j, m[16] - 1)

    grid_spec = pltpu.PrefetchScalarGridSpec(
        num_scalar_prefetch=1,
        grid=(MAX_TILES, NS),
        in_specs=[
            pl.BlockSpec((TM, KC),
                         lambda j, s, m: (jpin(j, s, m), jnp.minimum(s, KA - 1))),
            pl.BlockSpec((1, KC, DFF),
                         lambda j, s, m: (m[j], jnp.minimum(s, KA - 1), 0)),
            pl.BlockSpec((1, 1, DFF), lambda j, s, m: (m[j], 0, 0)),
            pl.BlockSpec((1, TFB, C),
                         lambda j, s, m: (m[j], jnp.maximum(s - KA, 0), 0)),
            pl.BlockSpec((1, 1, C), lambda j, s, m: (m[j], 0, 0)),
        ],
        out_specs=pl.BlockSpec((TM, C), lambda j, s, m: (j, 0)),
        scratch_shapes=[pltpu.VMEM((TM, DFF), jnp.float32)],
    )
    return pl.pallas_call(
        _ffn_body,
        grid_spec=grid_spec,
        out_shape=jax.ShapeDtypeStruct((ROWS, C), jnp.float32),
        compiler_params=pltpu.CompilerParams(
            dimension_semantics=("arbitrary", "arbitrary")),
    )(meta, xs, Wfc, bfc.reshape(E, 1, DFF), Wproj, bproj.reshape(E, 1, C))


def _gather_body(y_hbm, pos0_hbm, pos1_hbm, yc0_hbm, yc1_hbm, rows, idx, sem):
    cid = lax.axis_index("c")
    sid = lax.axis_index("s")
    w = sid * 2 + cid
    tpw = T // 32
    for chunk in range(2):
        base = w * tpw + chunk * 32
        pltpu.sync_copy(pos0_hbm.at[pl.ds(base, 32)], idx)
        pltpu.async_copy(y_hbm.at[idx], rows, sem).wait()
        pltpu.sync_copy(rows, yc0_hbm.at[pl.ds(base, 32)])
        pltpu.sync_copy(pos1_hbm.at[pl.ds(base, 32)], idx)
        pltpu.async_copy(y_hbm.at[idx], rows, sem).wait()
        pltpu.sync_copy(rows, yc1_hbm.at[pl.ds(base, 32)])


def _gather(y, pos0, pos1):
    mesh = plsc.VectorSubcoreMesh(core_axis_name="c", subcore_axis_name="s")
    kern = functools.partial(
        pl.kernel,
        out_type=(
            jax.ShapeDtypeStruct((T, C), jnp.float32),
            jax.ShapeDtypeStruct((T, C), jnp.float32),
        ),
        mesh=mesh,
        scratch_types=[
            pltpu.VMEM((32, C), jnp.float32),
            pltpu.VMEM((32,), jnp.int32),
            pltpu.SemaphoreType.DMA,
        ],
    )(_gather_body)
    return kern(y, pos0, pos1)


def _combine_body(y0_ref, y1_ref, g_ref, o_ref):
    g = g_ref[...]
    o_ref[...] = y0_ref[...] * g[:, 0:1] + y1_ref[...] * g[:, 1:2]


def _combine(yc0, yc1, gates):
    grid_m = T // TM
    return pl.pallas_call(
        _combine_body,
        grid=(grid_m,),
        out_shape=jax.ShapeDtypeStruct((T, C), jnp.float32),
        in_specs=[
            pl.BlockSpec((TM, C), lambda m: (m, 0)),
            pl.BlockSpec((TM, C), lambda m: (m, 0)),
            pl.BlockSpec((TM, 2), lambda m: (m, 0)),
        ],
        out_specs=pl.BlockSpec((TM, C), lambda m: (m, 0)),
    )(yc0, yc1, gates)


def kernel(x, Wg, bg, Wfc, bfc, Wproj, bproj):
    Bb, Tt, Cc = x.shape
    x_flat = x.reshape(Tt, Cc)

    eids, gates = _router(x_flat, Wg, bg)
    pos0, pos1, meta = _sortmeta(eids.reshape(PAIRS))
    xs = _dispatch(x_flat, pos0, pos1)
    y = _ffn(meta, xs, Wfc, bfc, Wproj, bproj)
    yc0, yc1 = _gather(y, pos0, pos1)
    out = _combine(yc0, yc1, gates)
    return out.reshape(Bb, Tt, Cc)


# speedup vs baseline: 1.3486x; 1.3486x over previous
import dataclasses
import functools

import jax
import jax.numpy as jnp
from jax import lax
from jax.experimental import pallas as pl
from jax.experimental.pallas import tpu as pltpu
from jax.experimental.pallas import tpu_sc as plsc

T = 2048
C = 2048
DFF = 8192
E = 8
TM = 512
TF = 1024
GRID_F = DFF // TF
MAX_TILES = 16
ROWS = MAX_TILES * TM
NW = 16
PAIRS = 2 * T
PPW = PAIRS // NW
VPW = PPW // 16


def _router_body(x_ref, wg_ref, bg_ref, eids_ref, gates_ref):
    s = jnp.dot(x_ref[...], wg_ref[...], preferred_element_type=jnp.float32)
    s = s + bg_ref[...]
    lane = lax.broadcasted_iota(jnp.int32, s.shape, 1)
    m1 = jnp.max(s, axis=1, keepdims=True)
    e1 = jnp.min(jnp.where(s == m1, lane, E), axis=1, keepdims=True)
    s2 = jnp.where(lane == e1, -jnp.inf, s)
    m2 = jnp.max(s2, axis=1, keepdims=True)
    e2 = jnp.min(jnp.where(s2 == m2, lane, E), axis=1, keepdims=True)
    g2 = 1.0 / (1.0 + jnp.exp(m1 - m2))
    g1 = 1.0 - g2
    eids_ref[...] = jnp.concatenate([e1, e2], axis=1).astype(jnp.int32)
    gates_ref[...] = jnp.concatenate([g1, g2], axis=1)


def _router(x_flat, Wg, bg):
    return pl.pallas_call(
        _router_body,
        out_shape=(
            jax.ShapeDtypeStruct((T, 2), jnp.int32),
            jax.ShapeDtypeStruct((T, 2), jnp.float32),
        ),
        in_specs=[
            pl.BlockSpec((T, C), lambda: (0, 0)),
            pl.BlockSpec((C, E), lambda: (0, 0)),
            pl.BlockSpec((1, E), lambda: (0, 0)),
        ],
        out_specs=(
            pl.BlockSpec((T, 2), lambda: (0, 0)),
            pl.BlockSpec((T, 2), lambda: (0, 0)),
        ),
    )(x_flat, Wg, bg.reshape(1, E))


def _sortmeta_body(eids_hbm, pos0_hbm, pos1_hbm, meta_hbm,
                   eidv, rankv_m, histv, allh, shh, startv, outv):
    cid = lax.axis_index("c")
    wid = lax.axis_index("s")
    iota = lax.iota(jnp.int32, 16)

    @pl.when(cid == 0)
    def _work():
        pltpu.sync_copy(eids_hbm.at[pl.ds(wid * PPW, PPW)], eidv)

        def vbody(v, cnts):
            ev = eidv[pl.ds(v * 16, 16)]
            rk = jnp.zeros((16,), jnp.int32)
            new = []
            for e in range(E):
                m = ev == e
                mi = m.astype(jnp.int32)
                pref = plsc.cumsum(mi)
                tot = jnp.sum(mi)
                rk = jnp.where(m, cnts[e] + pref - 1, rk)
                new.append(cnts[e] + tot)
            rankv_m[pl.ds(v * 16, 16)] = rk
            return tuple(new)

        cnts = lax.fori_loop(0, VPW, vbody,
                             tuple(jnp.int32(0) for _ in range(E)))

        hv = jnp.zeros((16,), jnp.int32)
        for e in range(E):
            hv = jnp.where(iota == e, cnts[e], hv)
        histv[...] = hv
        pltpu.sync_copy(histv, shh.at[wid])
        plsc.subcore_barrier()

        pltpu.sync_copy(shh, allh)
        base = jnp.zeros((16,), jnp.int32)
        tot = jnp.zeros((16,), jnp.int32)
        for j in range(NW):
            hj = allh[j, :]
            base = base + jnp.where(j < wid, hj, 0)
            tot = tot + hj
        padded = ((tot + (TM - 1)) >> 9) << 9
        o_incl = plsc.cumsum(padded)
        o_excl = o_incl - padded
        startv[...] = o_excl + base

        def pbody(v, _):
            ev = eidv[pl.ds(v * 16, 16)]
            st = plsc.load_gather(startv, [ev])
            rankv_m[pl.ds(v * 16, 16)] = st + rankv_m[pl.ds(v * 16, 16)]
            return 0

        lax.fori_loop(0, VPW, pbody, 0)

        half = PPW // 2
        for u in range(VPW // 2):
            idx0 = u * 32 + 2 * iota
            outv[pl.ds(u * 16, 16)] = plsc.load_gather(rankv_m, [idx0])
            outv[pl.ds(half + u * 16, 16)] = plsc.load_gather(rankv_m, [idx0 + 1])
        pltpu.sync_copy(outv.at[pl.ds(0, half)],
                        pos0_hbm.at[pl.ds(wid * half, half)])
        pltpu.sync_copy(outv.at[pl.ds(half, half)],
                        pos1_hbm.at[pl.ds(wid * half, half)])

        @pl.when(wid == 0)
        def _meta():
            tev = jnp.zeros((16,), jnp.int32)
            for e in range(E):
                oe = jnp.sum(jnp.where(iota == e, o_incl, 0))
                tev = tev + (oe <= iota * TM).astype(jnp.int32)
            last_e = jnp.max(jnp.where(tot > 0, iota, 0))
            tev = jnp.minimum(tev, last_e)
            nt = jnp.sum(padded) >> 9
            histv[...] = tev
            pltpu.sync_copy(histv, meta_hbm.at[pl.ds(0, 16)])
            histv[...] = jnp.zeros((16,), jnp.int32) + nt
            pltpu.sync_copy(histv, meta_hbm.at[pl.ds(16, 16)])


def _sc_compiler_params():
    cp = pltpu.CompilerParams()
    if "needs_layout_passes" in pltpu.CompilerParams.__dataclass_fields__:
        cp = dataclasses.replace(cp, needs_layout_passes=False)
    return cp


def _sortmeta(eids_flat):
    mesh = plsc.VectorSubcoreMesh(core_axis_name="c", subcore_axis_name="s")
    kern = functools.partial(
        pl.kernel,
        compiler_params=_sc_compiler_params(),
        out_type=(
            jax.ShapeDtypeStruct((T,), jnp.int32),
            jax.ShapeDtypeStruct((T,), jnp.int32),
            jax.ShapeDtypeStruct((32,), jnp.int32),
        ),
        mesh=mesh,
        scratch_types=[
            pltpu.VMEM((PPW,), jnp.int32),
            pltpu.VMEM((PPW,), jnp.int32),
            pltpu.VMEM((16,), jnp.int32),
            pltpu.VMEM((NW, 16), jnp.int32),
            pltpu.VMEM_SHARED((NW, 16), jnp.int32),
            pltpu.VMEM((16,), jnp.int32),
            pltpu.VMEM((PPW,), jnp.int32),
        ],
    )(_sortmeta_body)
    return kern(eids_flat)


def _dispatch_body(x_hbm, pos0_hbm, pos1_hbm, xs_hbm, rows, i0, i1, sem):
    cid = lax.axis_index("c")
    sid = lax.axis_index("s")
    w = sid * 2 + cid
    tpw = T // 32
    for chunk in range(2):
        base = w * tpw + chunk * 32
        pltpu.sync_copy(x_hbm.at[pl.ds(base, 32)], rows)
        pltpu.sync_copy(pos0_hbm.at[pl.ds(base, 32)], i0)
        pltpu.sync_copy(pos1_hbm.at[pl.ds(base, 32)], i1)
        pltpu.async_copy(rows, xs_hbm.at[i0], sem).wait()
        pltpu.async_copy(rows, xs_hbm.at[i1], sem).wait()


def _dispatch(x_flat, pos0, pos1):
    mesh = plsc.VectorSubcoreMesh(core_axis_name="c", subcore_axis_name="s")
    kern = functools.partial(
        pl.kernel,
        out_type=jax.ShapeDtypeStruct((ROWS, C), jnp.float32),
        mesh=mesh,
        scratch_types=[
            pltpu.VMEM((32, C), jnp.float32),
            pltpu.VMEM((32,), jnp.int32),
            pltpu.VMEM((32,), jnp.int32),
            pltpu.SemaphoreType.DMA,
        ],
    )(_dispatch_body)
    return kern(x_flat, pos0, pos1)


def _ffn_body(meta_ref, xs_ref, wfc_ref, bfc_ref, wproj_ref, bproj_ref, o_ref):
    j = pl.program_id(0)
    f = pl.program_id(1)
    nt = meta_ref[16]

    @pl.when(j < nt)
    def _active():
        h = jnp.dot(xs_ref[...], wfc_ref[0], preferred_element_type=jnp.float32)
        h = h + bfc_ref[0]
        h = 0.5 * h * (1.0 + jnp.tanh(0.7978845608028654
                                      * (h + 0.044715 * h * h * h)))
        yb = jnp.dot(h, wproj_ref[0], preferred_element_type=jnp.float32)

        @pl.when(f == 0)
        def _():
            o_ref[...] = yb + bproj_ref[0]

        @pl.when(f != 0)
        def _():
            o_ref[...] = o_ref[...] + yb

    @pl.when(jnp.logical_and(j >= nt, f == 0))
    def _inactive():
        o_ref[...] = jnp.zeros_like(o_ref)


def _ffn(meta, xs, Wfc, bfc, Wproj, bproj):
    def jpin(j, f, m):
        return jnp.minimum(j, m[16] - 1)

    def fpin(j, f, m):
        return jnp.where(j < m[16], f, GRID_F - 1)

    grid_spec = pltpu.PrefetchScalarGridSpec(
        num_scalar_prefetch=1,
        grid=(MAX_TILES, GRID_F),
        in_specs=[
            pl.BlockSpec((TM, C), lambda j, f, m: (jpin(j, f, m), 0)),
            pl.BlockSpec((1, C, TF), lambda j, f, m: (m[j], 0, fpin(j, f, m))),
            pl.BlockSpec((1, 1, TF), lambda j, f, m: (m[j], 0, fpin(j, f, m))),
            pl.BlockSpec((1, TF, C), lambda j, f, m: (m[j], fpin(j, f, m), 0)),
            pl.BlockSpec((1, 1, C), lambda j, f, m: (m[j], 0, 0)),
        ],
        out_specs=pl.BlockSpec((TM, C), lambda j, f, m: (j, 0)),
    )
    return pl.pallas_call(
        _ffn_body,
        grid_spec=grid_spec,
        out_shape=jax.ShapeDtypeStruct((ROWS, C), jnp.float32),
        compiler_params=pltpu.CompilerParams(
            dimension_semantics=("arbitrary", "arbitrary")),
    )(meta, xs, Wfc, bfc.reshape(E, 1, DFF), Wproj, bproj.reshape(E, 1, C))


def _gather_body(y_hbm, pos0_hbm, pos1_hbm, yc0_hbm, yc1_hbm, rows, idx, sem):
    cid = lax.axis_index("c")
    sid = lax.axis_index("s")
    w = sid * 2 + cid
    tpw = T // 32
    for chunk in range(2):
        base = w * tpw + chunk * 32
        pltpu.sync_copy(pos0_hbm.at[pl.ds(base, 32)], idx)
        pltpu.async_copy(y_hbm.at[idx], rows, sem).wait()
        pltpu.sync_copy(rows, yc0_hbm.at[pl.ds(base, 32)])
        pltpu.sync_copy(pos1_hbm.at[pl.ds(base, 32)], idx)
        pltpu.async_copy(y_hbm.at[idx], rows, sem).wait()
        pltpu.sync_copy(rows, yc1_hbm.at[pl.ds(base, 32)])


def _gather(y, pos0, pos1):
    mesh = plsc.VectorSubcoreMesh(core_axis_name="c", subcore_axis_name="s")
    kern = functools.partial(
        pl.kernel,
        out_type=(
            jax.ShapeDtypeStruct((T, C), jnp.float32),
            jax.ShapeDtypeStruct((T, C), jnp.float32),
        ),
        mesh=mesh,
        scratch_types=[
            pltpu.VMEM((32, C), jnp.float32),
            pltpu.VMEM((32,), jnp.int32),
            pltpu.SemaphoreType.DMA,
        ],
    )(_gather_body)
    return kern(y, pos0, pos1)


def _combine_body(y0_ref, y1_ref, g_ref, o_ref):
    g = g_ref[...]
    o_ref[...] = y0_ref[...] * g[:, 0:1] + y1_ref[...] * g[:, 1:2]


def _combine(yc0, yc1, gates):
    grid_m = T // TM
    return pl.pallas_call(
        _combine_body,
        grid=(grid_m,),
        out_shape=jax.ShapeDtypeStruct((T, C), jnp.float32),
        in_specs=[
            pl.BlockSpec((TM, C), lambda m: (m, 0)),
            pl.BlockSpec((TM, C), lambda m: (m, 0)),
            pl.BlockSpec((TM, 2), lambda m: (m, 0)),
        ],
        out_specs=pl.BlockSpec((TM, C), lambda m: (m, 0)),
    )(yc0, yc1, gates)


def kernel(x, Wg, bg, Wfc, bfc, Wproj, bproj):
    Bb, Tt, Cc = x.shape
    x_flat = x.reshape(Tt, Cc)

    eids, gates = _router(x_flat, Wg, bg)
    pos0, pos1, meta = _sortmeta(eids.reshape(PAIRS))
    xs = _dispatch(x_flat, pos0, pos1)
    y = _ffn(meta, xs, Wfc, bfc, Wproj, bproj)
    yc0, yc1 = _gather(y, pos0, pos1)
    out = _combine(yc0, yc1, gates)
    return out.reshape(Bb, Tt, Cc)
